# baseline (speedup 1.0000x reference)
import functools

import jax
import jax.numpy as jnp
from jax import lax
from jax.experimental import pallas as pl
from jax.experimental.pallas import tpu as pltpu
from jax.experimental.pallas import tpu_sc as plsc

_NC = 2
_NS = 16
_NW = _NC * _NS
_L = 16
_NBUF = 3


def _wid():
    return lax.axis_index("s") * _NC + lax.axis_index("c")


@functools.lru_cache(maxsize=None)
def _build_pair_table(V, D):
    VP = V // 2
    n_full = V // (2 * D)
    tail = V - n_full * 2 * D
    u_per_w = (n_full + _NW - 1) // _NW
    mesh = plsc.VectorSubcoreMesh(core_axis_name="c", subcore_axis_name="s")

    @functools.partial(
        pl.kernel,
        out_type=jax.ShapeDtypeStruct((VP, 2 * D), jnp.float32),
        mesh=mesh,
        scratch_types=[
            pltpu.VMEM((_NBUF, D, 2 * D), jnp.float32),
            pltpu.VMEM((_NBUF, D, 2 * D), jnp.float32),
            pltpu.SemaphoreType.DMA((_NBUF,)),
            pltpu.SemaphoreType.DMA((_NBUF,)),
        ],
        compiler_params=pltpu.CompilerParams(needs_layout_passes=False),
    )
    def k(tab_t, tailp, ptab, stg_v, pair_v, sem_in, sem_out):
        wid = _wid()
        u0 = wid * u_per_w

        def start_read(c, b):
            pltpu.async_copy(
                tab_t.at[:, pl.ds(c * 2 * D, 2 * D)], stg_v.at[b], sem_in.at[b])

        def wait_read(b):
            pltpu.make_async_copy(
                tab_t.at[:, pl.ds(0, 2 * D)], stg_v.at[b], sem_in.at[b]).wait()

        def transpose_unit(b):
            def body(p, carry):
                for t in range(2 * D // _L):
                    rows = lax.iota(jnp.int32, _L) + (t * _L) % D
                    col = jnp.full((_L,), 2 * p + (t * _L) // D, jnp.int32)
                    v = plsc.load_gather(stg_v.at[b], [rows, col])
                    pair_v[b, p, pl.ds(t * _L, _L)] = v
                return carry
            lax.fori_loop(0, D, body, 0)

        def start_write(c, b):
            pltpu.async_copy(
                pair_v.at[b], ptab.at[pl.ds(c * D, D)], sem_out.at[b])

        def wait_write(b):
            pltpu.make_async_copy(
                pair_v.at[b], ptab.at[pl.ds(0, D)], sem_out.at[b]).wait()

        n_mine = jnp.minimum(
            jnp.maximum(n_full - u0, 0), u_per_w).astype(jnp.int32)

        for b in range(_NBUF):
            @pl.when(b < n_mine)
            def _():
                start_read(u0 + b, b)

        def step(j, carry):
            b = lax.rem(j, _NBUF)
            wait_read(b)

            @pl.when(j >= _NBUF)
            def _():
                wait_write(b)

            transpose_unit(b)
            start_write(u0 + j, b)

            @pl.when(j + _NBUF < n_mine)
            def _():
                start_read(u0 + j + _NBUF, b)

            return carry

        lax.fori_loop(0, n_mine, step, 0)
        for b in range(_NBUF):
            @pl.when(b < n_mine)
            def _():
                wait_write(b)

        if tail:
            tp = tail // 2
            @pl.when(wid == _NW - 1)
            def _():
                pltpu.async_copy(
                    tailp, stg_v.at[0, pl.ds(0, tp)], sem_in.at[0])
                pltpu.make_async_copy(
                    tailp, stg_v.at[0, pl.ds(0, tp)], sem_in.at[0]).wait()
                pltpu.async_copy(
                    stg_v.at[0, pl.ds(0, tp)],
                    ptab.at[pl.ds(n_full * D, tp)], sem_out.at[0])
                pltpu.make_async_copy(
                    stg_v.at[0, pl.ds(0, tp)],
                    ptab.at[pl.ds(0, tp)], sem_out.at[0]).wait()

    return k


@functools.lru_cache(maxsize=None)
def _build_gather(BATCH, HIST, V, D):
    VP = V // 2
    BT = 128
    n_bt = BATCH // BT
    n_units = n_bt * HIST
    u_per_w = n_units // _NW
    mesh = plsc.VectorSubcoreMesh(core_axis_name="c", subcore_axis_name="s")

    @functools.partial(
        pl.kernel,
        out_type=jax.ShapeDtypeStruct((HIST, D, BATCH), jnp.float32),
        mesh=mesh,
        scratch_types=[
            pltpu.VMEM((_NBUF, BT), jnp.int32),
            pltpu.VMEM((_NBUF, BT), jnp.int32),
            pltpu.VMEM((_NBUF, BT, 2 * D), jnp.float32),
            pltpu.VMEM((_NBUF, D, BT), jnp.float32),
            pltpu.SemaphoreType.DMA((_NBUF,)),
            pltpu.SemaphoreType.DMA((_NBUF,)),
            pltpu.SemaphoreType.DMA((_NBUF,)),
        ],
        compiler_params=pltpu.CompilerParams(needs_layout_passes=False),
    )
    def k(idx_t, ptab, out, idx_v, pidx_v, rows_v, blk_v,
          sem_idx, sem_in, sem_out):
        wid = _wid()
        u0 = wid * u_per_w

        def hb(u):
            return u // n_bt, lax.rem(u, n_bt)

        def start_idx(u, b):
            h, bt = hb(u)
            pltpu.async_copy(
                idx_t.at[h, pl.ds(bt * BT, BT)], idx_v.at[b], sem_idx.at[b])

        def start_gather(b):
            pltpu.make_async_copy(
                idx_t.at[0, pl.ds(0, BT)], idx_v.at[b], sem_idx.at[b]).wait()
            for t in range(BT // _L):
                raw = idx_v[b, pl.ds(t * _L, _L)]
                pidx_v[b, pl.ds(t * _L, _L)] = lax.shift_right_logical(raw, 1)
            pltpu.async_copy(ptab.at[pidx_v.at[b]], rows_v.at[b], sem_in.at[b])

        def finish_unit(u, b):
            h, bt = hb(u)
            pltpu.make_async_copy(
                ptab.at[pidx_v.at[b]], rows_v.at[b], sem_in.at[b]).wait()

            def body(d, carry):
                for t in range(BT // _L):
                    raw = idx_v[b, pl.ds(t * _L, _L)]
                    cols = (raw & 1) * D + d
                    rows = lax.iota(jnp.int32, _L) + t * _L
                    v = plsc.load_gather(rows_v.at[b], [rows, cols])
                    blk_v[b, d, pl.ds(t * _L, _L)] = v
                return carry
            lax.fori_loop(0, D, body, 0)
            pltpu.async_copy(
                blk_v.at[b], out.at[h, :, pl.ds(bt * BT, BT)], sem_out.at[b])

        def wait_out(b):
            pltpu.make_async_copy(
                blk_v.at[b], out.at[0, :, pl.ds(0, BT)], sem_out.at[b]).wait()

        for b in range(_NBUF):
            start_idx(u0 + b, b)
        for b in range(_NBUF):
            start_gather(b)

        def step(j, carry):
            b = lax.rem(j, _NBUF)

            @pl.when(j >= _NBUF)
            def _():
                wait_out(b)

            finish_unit(u0 + j, b)

            @pl.when(j + _NBUF < u_per_w)
            def _():
                start_idx(u0 + j + _NBUF, b)
                start_gather(b)

            return carry

        lax.fori_loop(0, u_per_w, step, 0)
        for b in range(_NBUF):
            wait_out(b)

    return k


def kernel(indices, table):
    BATCH, HIST = indices.shape
    V, D = table.shape
    idx_t = jnp.transpose(indices).astype(jnp.int32)
    tab_t = jnp.transpose(table)
    n_full = V // (2 * D)
    tailp = table[n_full * 2 * D:].reshape(-1, 2 * D)
    ptab = _build_pair_table(V, D)(tab_t, tailp)
    outp = _build_gather(BATCH, HIST, V, D)(idx_t, ptab)
    return jnp.transpose(outp, (2, 0, 1))


# speedup vs baseline: 1.8735x; 1.8735x over previous
import functools

import jax
import jax.numpy as jnp
from jax import lax
from jax.experimental import pallas as pl
from jax.experimental.pallas import tpu as pltpu
from jax.experimental.pallas import tpu_sc as plsc

_NC = 2
_NS = 16
_NW = _NC * _NS
_L = 16
_NBUF = 3
_CB = 512


def _split_point(V):
    n_blk = pl.cdiv(V, _CB)
    return ((n_blk + 1) // 2) * _CB


@functools.lru_cache(maxsize=None)
def _build_pair_table(V, D):
    S = _split_point(V)
    n_half = S // _CB

    def body(xa_ref, xb_ref, o_ref):
        o_ref[...] = jnp.concatenate([xa_ref[...].T, xb_ref[...].T], axis=1)

    return pl.pallas_call(
        body,
        grid=(n_half,),
        in_specs=[
            pl.BlockSpec((D, _CB), lambda i: (0, i)),
            pl.BlockSpec((D, _CB), lambda i: (0, i + n_half)),
        ],
        out_specs=pl.BlockSpec((_CB, 2 * D), lambda i: (i, 0)),
        out_shape=jax.ShapeDtypeStruct((S, 2 * D), jnp.float32),
    )


@functools.lru_cache(maxsize=None)
def _build_gather(BATCH, HIST, V, D):
    S = _split_point(V)
    BT = 128
    n_bt = BATCH // BT
    n_units = n_bt * HIST
    u_per_w = n_units // _NW
    mesh = plsc.VectorSubcoreMesh(core_axis_name="c", subcore_axis_name="s")

    @functools.partial(
        pl.kernel,
        out_type=jax.ShapeDtypeStruct((HIST, D, BATCH), jnp.float32),
        mesh=mesh,
        scratch_types=[
            pltpu.VMEM((_NBUF, BT), jnp.int32),
            pltpu.VMEM((_NBUF, BT), jnp.int32),
            pltpu.VMEM((_NBUF, BT, 2 * D), jnp.float32),
            pltpu.VMEM((_NBUF, D, BT), jnp.float32),
            pltpu.SemaphoreType.DMA((_NBUF,)),
            pltpu.SemaphoreType.DMA((_NBUF,)),
            pltpu.SemaphoreType.DMA((_NBUF,)),
        ],
        compiler_params=pltpu.CompilerParams(needs_layout_passes=False),
    )
    def k(idx_t, ptab, out, idx_v, pidx_v, rows_v, blk_v,
          sem_idx, sem_in, sem_out):
        wid = lax.axis_index("s") * _NC + lax.axis_index("c")
        u0 = wid * u_per_w

        def hb(u):
            return u // n_bt, lax.rem(u, n_bt)

        def start_idx(u, b):
            h, bt = hb(u)
            pltpu.async_copy(
                idx_t.at[h, pl.ds(bt * BT, BT)], idx_v.at[b], sem_idx.at[b])

        def start_gather(b):
            pltpu.make_async_copy(
                idx_t.at[0, pl.ds(0, BT)], idx_v.at[b], sem_idx.at[b]).wait()
            for t in range(BT // _L):
                raw = idx_v[b, pl.ds(t * _L, _L)]
                pidx_v[b, pl.ds(t * _L, _L)] = jnp.where(raw >= S, raw - S, raw)
            pltpu.async_copy(ptab.at[pidx_v.at[b]], rows_v.at[b], sem_in.at[b])

        def finish_unit(u, b):
            h, bt = hb(u)
            pltpu.make_async_copy(
                ptab.at[pidx_v.at[b]], rows_v.at[b], sem_in.at[b]).wait()

            rows_l = [lax.iota(jnp.int32, _L) + t * _L for t in range(BT // _L)]
            par_l = [jnp.where(idx_v[b, pl.ds(t * _L, _L)] >= S, D, 0)
                     for t in range(BT // _L)]

            def body(d, carry):
                dvec = lax.broadcast_in_dim(d, (_L,), ())
                for t in range(BT // _L):
                    v = plsc.load_gather(
                        rows_v.at[b], [rows_l[t], par_l[t] + dvec])
                    blk_v[b, d, pl.ds(t * _L, _L)] = v
                return carry

            lax.fori_loop(0, D, body, 0)
            pltpu.async_copy(
                blk_v.at[b], out.at[h, :, pl.ds(bt * BT, BT)], sem_out.at[b])

        def wait_out(b):
            pltpu.make_async_copy(
                blk_v.at[b], out.at[0, :, pl.ds(0, BT)], sem_out.at[b]).wait()

        for b in range(_NBUF):
            start_idx(u0 + b, b)
        for b in range(_NBUF):
            start_gather(b)

        def step(j, carry):
            b = lax.rem(j, _NBUF)

            @pl.when(j >= _NBUF)
            def _():
                wait_out(b)

            finish_unit(u0 + j, b)

            @pl.when(j + _NBUF < u_per_w)
            def _():
                start_idx(u0 + j + _NBUF, b)
                start_gather(b)

            return carry

        lax.fori_loop(0, u_per_w, step, 0)
        for b in range(_NBUF):
            wait_out(b)

    return k


def kernel(indices, table):
    BATCH, HIST = indices.shape
    V, D = table.shape
    idx_t = jnp.transpose(indices).astype(jnp.int32)
    tab_t = jnp.transpose(table)
    ptab = _build_pair_table(V, D)(tab_t, tab_t)
    outp = _build_gather(BATCH, HIST, V, D)(idx_t, ptab)
    return jnp.transpose(outp, (2, 0, 1))


# speedup vs baseline: 3.4842x; 1.8597x over previous
import functools

import jax
import jax.numpy as jnp
from jax import lax
from jax.experimental import pallas as pl
from jax.experimental.pallas import tpu as pltpu
from jax.experimental.pallas import tpu_sc as plsc

_NC = 2
_NS = 16
_NW = _NC * _NS
_L = 16
_NBUF = 3
_CB = 2048


def _split_point(V):
    return (V // (2 * _CB)) * _CB


@functools.lru_cache(maxsize=None)
def _build_pair_table(V, D):
    S = _split_point(V)
    R = V - S
    off = S // _CB

    def body(xa_ref, xb_ref, o_ref):
        o_ref[...] = jnp.concatenate([xa_ref[...].T, xb_ref[...].T], axis=1)

    return pl.pallas_call(
        body,
        grid=(pl.cdiv(R, _CB),),
        in_specs=[
            pl.BlockSpec((D, _CB), lambda i: (0, i)),
            pl.BlockSpec((D, _CB), lambda i: (0, i + off)),
        ],
        out_specs=pl.BlockSpec((_CB, 2 * D), lambda i: (i, 0)),
        out_shape=jax.ShapeDtypeStruct((R, 2 * D), jnp.float32),
    )


@functools.lru_cache(maxsize=None)
def _build_gather(BATCH, HIST, V, D):
    S = _split_point(V)
    BT = 128
    n_bt = BATCH // BT
    n_units = n_bt * HIST
    u_per_w = n_units // _NW
    mesh = plsc.VectorSubcoreMesh(core_axis_name="c", subcore_axis_name="s")

    @functools.partial(
        pl.kernel,
        out_type=jax.ShapeDtypeStruct((HIST, D, BATCH), jnp.float32),
        mesh=mesh,
        scratch_types=[
            pltpu.VMEM((_NBUF, BT), jnp.int32),
            pltpu.VMEM((_NBUF, BT), jnp.int32),
            pltpu.VMEM((_NBUF, BT, 2 * D), jnp.float32),
            pltpu.VMEM((_NBUF, D, BT), jnp.float32),
            pltpu.SemaphoreType.DMA((_NBUF,)),
            pltpu.SemaphoreType.DMA((_NBUF,)),
            pltpu.SemaphoreType.DMA((_NBUF,)),
        ],
        compiler_params=pltpu.CompilerParams(needs_layout_passes=False),
    )
    def k(idx_t, ptab, out, idx_v, pidx_v, rows_v, blk_v,
          sem_idx, sem_in, sem_out):
        wid = lax.axis_index("s") * _NC + lax.axis_index("c")
        u0 = wid * u_per_w

        def hb(u):
            return u // n_bt, lax.rem(u, n_bt)

        def start_idx(u, b):
            h, bt = hb(u)
            pltpu.async_copy(
                idx_t.at[h, pl.ds(bt * BT, BT)], idx_v.at[b], sem_idx.at[b])

        def start_gather(b):
            pltpu.make_async_copy(
                idx_t.at[0, pl.ds(0, BT)], idx_v.at[b], sem_idx.at[b]).wait()
            for t in range(BT // _L):
                raw = idx_v[b, pl.ds(t * _L, _L)]
                pidx_v[b, pl.ds(t * _L, _L)] = jnp.where(raw >= S, raw - S, raw)
            pltpu.async_copy(ptab.at[pidx_v.at[b]], rows_v.at[b], sem_in.at[b])

        def finish_unit(u, b):
            h, bt = hb(u)
            pltpu.make_async_copy(
                ptab.at[pidx_v.at[b]], rows_v.at[b], sem_in.at[b]).wait()

            rows_l = [lax.iota(jnp.int32, _L) + t * _L for t in range(BT // _L)]
            par_l = [jnp.where(idx_v[b, pl.ds(t * _L, _L)] >= S, D, 0)
                     for t in range(BT // _L)]

            @plsc.parallel_loop(0, D, unroll=4)
            def _(d):
                dvec = lax.broadcast_in_dim(d, (_L,), ())
                for t in range(BT // _L):
                    v = plsc.load_gather(
                        rows_v.at[b], [rows_l[t], par_l[t] + dvec])
                    blk_v[b, d, pl.ds(t * _L, _L)] = v
            pltpu.async_copy(
                blk_v.at[b], out.at[h, :, pl.ds(bt * BT, BT)], sem_out.at[b])

        def wait_out(b):
            pltpu.make_async_copy(
                blk_v.at[b], out.at[0, :, pl.ds(0, BT)], sem_out.at[b]).wait()

        for b in range(_NBUF):
            start_idx(u0 + b, b)
        for b in range(_NBUF):
            start_gather(b)

        def step(j, carry):
            b = lax.rem(j, _NBUF)

            @pl.when(j >= _NBUF)
            def _():
                wait_out(b)

            finish_unit(u0 + j, b)

            @pl.when(j + _NBUF < u_per_w)
            def _():
                start_idx(u0 + j + _NBUF, b)
                start_gather(b)

            return carry

        lax.fori_loop(0, u_per_w, step, 0)
        for b in range(_NBUF):
            wait_out(b)

    return k


def kernel(indices, table):
    BATCH, HIST = indices.shape
    V, D = table.shape
    idx_t = jnp.transpose(indices).astype(jnp.int32)
    tab_t = jnp.transpose(table)
    ptab = _build_pair_table(V, D)(tab_t, tab_t)
    outp = _build_gather(BATCH, HIST, V, D)(idx_t, ptab)
    return jnp.transpose(outp, (2, 0, 1))


# speedup vs baseline: 3.7205x; 1.0678x over previous
import functools

import jax
import jax.numpy as jnp
from jax import lax
from jax.experimental import pallas as pl
from jax.experimental.pallas import tpu as pltpu
from jax.experimental.pallas import tpu_sc as plsc

_NC = 2
_NS = 16
_NW = _NC * _NS
_L = 16
_NBUF = 4
_CB = 4096


def _split_point(V):
    return (V // (2 * _CB)) * _CB


@functools.lru_cache(maxsize=None)
def _build_pair_table(V, D):
    S = _split_point(V)
    R = V - S
    off = S // _CB

    def body(xa_ref, xb_ref, o_ref):
        o_ref[...] = jnp.concatenate([xa_ref[...].T, xb_ref[...].T], axis=1)

    return pl.pallas_call(
        body,
        grid=(pl.cdiv(R, _CB),),
        in_specs=[
            pl.BlockSpec((D, _CB), lambda i: (0, i)),
            pl.BlockSpec((D, _CB), lambda i: (0, i + off)),
        ],
        out_specs=pl.BlockSpec((_CB, 2 * D), lambda i: (i, 0)),
        out_shape=jax.ShapeDtypeStruct((R, 2 * D), jnp.float32),
    )


@functools.lru_cache(maxsize=None)
def _build_gather(BATCH, HIST, V, D):
    S = _split_point(V)
    BT = 128
    n_bt = BATCH // BT
    n_units = n_bt * HIST
    u_per_w = n_units // _NW
    mesh = plsc.VectorSubcoreMesh(core_axis_name="c", subcore_axis_name="s")

    @functools.partial(
        pl.kernel,
        out_type=jax.ShapeDtypeStruct((HIST, D, BATCH), jnp.float32),
        mesh=mesh,
        scratch_types=[
            pltpu.VMEM((_NBUF, BT), jnp.int32),
            pltpu.VMEM((_NBUF, BT), jnp.int32),
            pltpu.VMEM((_NBUF, BT, 2 * D), jnp.float32),
            pltpu.VMEM((_NBUF, D, BT), jnp.float32),
            pltpu.SemaphoreType.DMA((_NBUF,)),
            pltpu.SemaphoreType.DMA((_NBUF,)),
            pltpu.SemaphoreType.DMA((_NBUF,)),
        ],
        compiler_params=pltpu.CompilerParams(needs_layout_passes=False),
    )
    def k(idx_t, ptab, out, idx_v, pidx_v, rows_v, blk_v,
          sem_idx, sem_in, sem_out):
        wid = lax.axis_index("s") * _NC + lax.axis_index("c")
        u0 = wid * u_per_w

        def hb(u):
            return u // n_bt, lax.rem(u, n_bt)

        def start_idx(u, b):
            h, bt = hb(u)
            pltpu.async_copy(
                idx_t.at[h, pl.ds(bt * BT, BT)], idx_v.at[b], sem_idx.at[b])

        def start_gather(b):
            pltpu.make_async_copy(
                idx_t.at[0, pl.ds(0, BT)], idx_v.at[b], sem_idx.at[b]).wait()
            for t in range(BT // _L):
                raw = idx_v[b, pl.ds(t * _L, _L)]
                pidx_v[b, pl.ds(t * _L, _L)] = jnp.where(raw >= S, raw - S, raw)
            pltpu.async_copy(ptab.at[pidx_v.at[b]], rows_v.at[b], sem_in.at[b])

        def finish_unit(u, b):
            h, bt = hb(u)
            pltpu.make_async_copy(
                ptab.at[pidx_v.at[b]], rows_v.at[b], sem_in.at[b]).wait()

            rows_l = [lax.iota(jnp.int32, _L) + t * _L for t in range(BT // _L)]
            par_l = [jnp.where(idx_v[b, pl.ds(t * _L, _L)] >= S, D, 0)
                     for t in range(BT // _L)]

            @plsc.parallel_loop(0, D, unroll=8)
            def _(d):
                dvec = lax.broadcast_in_dim(d, (_L,), ())
                for t in range(BT // _L):
                    v = plsc.load_gather(
                        rows_v.at[b], [rows_l[t], par_l[t] + dvec])
                    blk_v[b, d, pl.ds(t * _L, _L)] = v
            pltpu.async_copy(
                blk_v.at[b], out.at[h, :, pl.ds(bt * BT, BT)], sem_out.at[b])

        def wait_out(b):
            pltpu.make_async_copy(
                blk_v.at[b], out.at[0, :, pl.ds(0, BT)], sem_out.at[b]).wait()

        for b in range(_NBUF):
            start_idx(u0 + b, b)
        for b in range(_NBUF):
            start_gather(b)

        def step(j, carry):
            b = lax.rem(j, _NBUF)

            @pl.when(j >= _NBUF)
            def _():
                wait_out(b)

            finish_unit(u0 + j, b)

            @pl.when(j + _NBUF < u_per_w)
            def _():
                start_idx(u0 + j + _NBUF, b)
                start_gather(b)

            return carry

        lax.fori_loop(0, u_per_w, step, 0)
        for b in range(_NBUF):
            wait_out(b)

    return k


def kernel(indices, table):
    BATCH, HIST = indices.shape
    V, D = table.shape
    idx_t = jnp.transpose(indices).astype(jnp.int32)
    tab_t = jnp.transpose(table)
    ptab = _build_pair_table(V, D)(tab_t, tab_t)
    outp = _build_gather(BATCH, HIST, V, D)(idx_t, ptab)
    return jnp.transpose(outp, (2, 0, 1))
